# baseline (speedup 1.0000x reference)
import functools

import jax
import jax.numpy as jnp
from jax import lax
from jax.experimental import pallas as pl
from jax.experimental.pallas import tpu as pltpu
from jax.experimental.pallas import tpu_sc as plsc

_CH = 16
_ZROWS = 8


_NBUF = 16


def _sc_aggregate(x, src, dst):
  n, d = x.shape
  e = src.shape[0]
  info = plsc.get_sparse_core_info()
  nc, ns = info.num_cores, info.num_subcores
  e_per_w = e // (nc * ns)
  n_chunks = e_per_w // _CH
  n_pad = -(-n // (8 * ns)) * (8 * ns)
  rows_per_s = n_pad // ns
  mesh = plsc.VectorSubcoreMesh(core_axis_name="c", subcore_axis_name="s")

  @functools.partial(
      pl.kernel,
      out_type=jax.ShapeDtypeStruct((nc, n_pad, d), jnp.float32),
      mesh=mesh,
      scratch_types=[
          pltpu.VMEM((e_per_w,), jnp.int32),
          pltpu.VMEM((_NBUF, _CH), jnp.int32),
          pltpu.VMEM((_NBUF, _CH, d), jnp.float32),
          pltpu.VMEM((_ZROWS, d), jnp.float32),
          pltpu.VMEM_SHARED((n_pad, d), jnp.float32),
          pltpu.SemaphoreType.DMA,
          pltpu.SemaphoreType.DMA,
          pltpu.SemaphoreType.DMA,
      ],
  )
  def k(x_hbm, src_hbm, dst_hbm, out_hbm, sidx_v, didx_v, rows_v, zb_v,
        acc_sh, sem_g, sem_i, sem_s):
    c = lax.axis_index("c")
    s = lax.axis_index("s")
    wid = c * ns + s
    base = wid * e_per_w

    pltpu.sync_copy(src_hbm.at[pl.ds(base, e_per_w)], sidx_v)

    def zb_body(i, _):
      zb_v[i // (d // 16), pl.ds((i % (d // 16)) * 16, 16)] = jnp.zeros(
          (16,), jnp.float32)
      return 0

    lax.fori_loop(0, _ZROWS * (d // 16), zb_body, 0)

    def zacc_body(t, _):
      pltpu.async_copy(zb_v, acc_sh.at[pl.ds(s * rows_per_s + t * _ZROWS,
                                             _ZROWS)], sem_s)
      return 0

    lax.fori_loop(0, rows_per_s // _ZROWS, zacc_body, 0)

    for j in range(_NBUF):
      pltpu.async_copy(dst_hbm.at[pl.ds(base + j * _CH, _CH)], didx_v.at[j],
                       sem_i)
      pltpu.async_copy(x_hbm.at[sidx_v.at[pl.ds(j * _CH, _CH)]], rows_v.at[j],
                       sem_g)

    def zdrain_body(t, _):
      pltpu.make_async_copy(zb_v, acc_sh.at[pl.ds(s * rows_per_s + t * _ZROWS,
                                                  _ZROWS)], sem_s).wait()
      return 0

    lax.fori_loop(0, rows_per_s // _ZROWS, zdrain_body, 0)
    plsc.subcore_barrier()

    def body(j, _):
      b = lax.rem(j, _NBUF)
      bp = lax.rem(j + _NBUF - 1, _NBUF)

      @pl.when(j > 0)
      def _():
        pltpu.make_async_copy(rows_v.at[bp], acc_sh.at[didx_v.at[bp]],
                              sem_s).wait()

      @pl.when(jnp.logical_and(j > 0, j + _NBUF - 1 < n_chunks))
      def _():
        off = (j + _NBUF - 1) * _CH
        pltpu.async_copy(x_hbm.at[sidx_v.at[pl.ds(off, _CH)]], rows_v.at[bp],
                         sem_g)
        pltpu.async_copy(dst_hbm.at[pl.ds(base + off, _CH)], didx_v.at[bp],
                         sem_i)

      pltpu.make_async_copy(dst_hbm.at[pl.ds(base, _CH)], didx_v.at[b],
                            sem_i).wait()
      pltpu.make_async_copy(x_hbm.at[sidx_v.at[pl.ds(j * _CH, _CH)]],
                            rows_v.at[b], sem_g).wait()
      pltpu.async_copy(rows_v.at[b], acc_sh.at[didx_v.at[b]], sem_s, add=True)
      return 0

    lax.fori_loop(0, n_chunks, body, 0)
    lastb = (n_chunks - 1) % _NBUF
    pltpu.make_async_copy(rows_v.at[lastb], acc_sh.at[didx_v.at[lastb]],
                          sem_s).wait()
    plsc.subcore_barrier()
    pltpu.sync_copy(acc_sh.at[pl.ds(s * rows_per_s, rows_per_s)],
                    out_hbm.at[c, pl.ds(s * rows_per_s, rows_per_s)])

  return k(x, src, dst)


def _tc_combine_matmul_relu(partials, weight, n):
  nc, _, d = partials.shape
  d_out = weight.shape[1]
  rows = 2000

  def body(p_ref, w_ref, o_ref):
    a = p_ref[0]
    for i in range(1, nc):
      a = a + p_ref[i]
    o_ref[...] = jnp.maximum(
        jnp.dot(a, w_ref[...], preferred_element_type=jnp.float32), 0.0)

  return pl.pallas_call(
      body,
      grid=(n // rows,),
      in_specs=[
          pl.BlockSpec((nc, rows, d), lambda i: (0, i, 0)),
          pl.BlockSpec((d, d_out), lambda i: (0, 0)),
      ],
      out_specs=pl.BlockSpec((rows, d_out), lambda i: (i, 0)),
      out_shape=jax.ShapeDtypeStruct((n, d_out), jnp.float32),
  )(partials, weight)


def kernel(input, adj, weight):
  src = adj[0].astype(jnp.int32)
  dst = adj[1].astype(jnp.int32)
  partials = _sc_aggregate(input, src, dst)
  return _tc_combine_matmul_relu(partials, weight, input.shape[0])


# speedup vs baseline: 1.1412x; 1.1412x over previous
import functools

import jax
import jax.numpy as jnp
from jax import lax
from jax.experimental import pallas as pl
from jax.experimental.pallas import tpu as pltpu
from jax.experimental.pallas import tpu_sc as plsc

_CH = 40
_ZROWS = 8


_NBUF = 7


def _sc_aggregate(x, src, dst):
  n, d = x.shape
  e = src.shape[0]
  info = plsc.get_sparse_core_info()
  nc, ns = info.num_cores, info.num_subcores
  e_per_w = e // (nc * ns)
  n_chunks = e_per_w // _CH
  n_pad = -(-n // (8 * ns)) * (8 * ns)
  rows_per_s = n_pad // ns
  mesh = plsc.VectorSubcoreMesh(core_axis_name="c", subcore_axis_name="s")

  @functools.partial(
      pl.kernel,
      out_type=jax.ShapeDtypeStruct((nc, n_pad, d), jnp.float32),
      mesh=mesh,
      scratch_types=[
          pltpu.VMEM((e_per_w,), jnp.int32),
          pltpu.VMEM((_NBUF, _CH), jnp.int32),
          pltpu.VMEM((_NBUF, _CH, d), jnp.float32),
          pltpu.VMEM((_ZROWS, d), jnp.float32),
          pltpu.VMEM_SHARED((n_pad, d), jnp.float32),
          pltpu.SemaphoreType.DMA,
          pltpu.SemaphoreType.DMA,
          pltpu.SemaphoreType.DMA,
      ],
  )
  def k(x_hbm, src_hbm, dst_hbm, out_hbm, sidx_v, didx_v, rows_v, zb_v,
        acc_sh, sem_g, sem_i, sem_s):
    c = lax.axis_index("c")
    s = lax.axis_index("s")
    wid = c * ns + s
    base = wid * e_per_w

    pltpu.async_copy(src_hbm.at[pl.ds(base, e_per_w)], sidx_v, sem_g)

    def zb_body(i, _):
      zb_v[i // (d // 16), pl.ds((i % (d // 16)) * 16, 16)] = jnp.zeros(
          (16,), jnp.float32)
      return 0

    lax.fori_loop(0, _ZROWS * (d // 16), zb_body, 0)

    def zacc_body(t, _):
      pltpu.async_copy(zb_v, acc_sh.at[pl.ds(s * rows_per_s + t * _ZROWS,
                                             _ZROWS)], sem_s)
      return 0

    lax.fori_loop(0, rows_per_s // _ZROWS, zacc_body, 0)

    for j in range(_NBUF):
      pltpu.async_copy(dst_hbm.at[pl.ds(base + j * _CH, _CH)], didx_v.at[j],
                       sem_i)
    pltpu.make_async_copy(src_hbm.at[pl.ds(base, e_per_w)], sidx_v,
                          sem_g).wait()
    for j in range(_NBUF):
      pltpu.async_copy(x_hbm.at[sidx_v.at[pl.ds(j * _CH, _CH)]], rows_v.at[j],
                       sem_g)

    def zdrain_body(t, _):
      pltpu.make_async_copy(zb_v, acc_sh.at[pl.ds(s * rows_per_s + t * _ZROWS,
                                                  _ZROWS)], sem_s).wait()
      return 0

    lax.fori_loop(0, rows_per_s // _ZROWS, zdrain_body, 0)
    plsc.subcore_barrier()

    def body(j, _):
      b = lax.rem(j, _NBUF)
      bp = lax.rem(j + _NBUF - 1, _NBUF)

      @pl.when(j > 0)
      def _():
        pltpu.make_async_copy(rows_v.at[bp], acc_sh.at[didx_v.at[bp]],
                              sem_s).wait()

      @pl.when(jnp.logical_and(j > 0, j + _NBUF - 1 < n_chunks))
      def _():
        off = (j + _NBUF - 1) * _CH
        pltpu.async_copy(x_hbm.at[sidx_v.at[pl.ds(off, _CH)]], rows_v.at[bp],
                         sem_g)
        pltpu.async_copy(dst_hbm.at[pl.ds(base + off, _CH)], didx_v.at[bp],
                         sem_i)

      pltpu.make_async_copy(dst_hbm.at[pl.ds(base, _CH)], didx_v.at[b],
                            sem_i).wait()
      pltpu.make_async_copy(x_hbm.at[sidx_v.at[pl.ds(j * _CH, _CH)]],
                            rows_v.at[b], sem_g).wait()
      pltpu.async_copy(rows_v.at[b], acc_sh.at[didx_v.at[b]], sem_s, add=True)
      return 0

    lax.fori_loop(0, n_chunks, body, 0)
    lastb = (n_chunks - 1) % _NBUF
    pltpu.make_async_copy(rows_v.at[lastb], acc_sh.at[didx_v.at[lastb]],
                          sem_s).wait()
    plsc.subcore_barrier()
    pltpu.sync_copy(acc_sh.at[pl.ds(s * rows_per_s, rows_per_s)],
                    out_hbm.at[c, pl.ds(s * rows_per_s, rows_per_s)])

  return k(x, src, dst)


def _tc_combine_matmul_relu(partials, weight, n):
  nc, _, d = partials.shape
  d_out = weight.shape[1]
  rows = 2000

  def body(p_ref, w_ref, o_ref):
    a = p_ref[0]
    for i in range(1, nc):
      a = a + p_ref[i]
    o_ref[...] = jnp.maximum(
        jnp.dot(a, w_ref[...], preferred_element_type=jnp.float32), 0.0)

  return pl.pallas_call(
      body,
      grid=(n // rows,),
      in_specs=[
          pl.BlockSpec((nc, rows, d), lambda i: (0, i, 0)),
          pl.BlockSpec((d, d_out), lambda i: (0, 0)),
      ],
      out_specs=pl.BlockSpec((rows, d_out), lambda i: (i, 0)),
      out_shape=jax.ShapeDtypeStruct((n, d_out), jnp.float32),
  )(partials, weight)


def kernel(input, adj, weight):
  src = adj[0].astype(jnp.int32)
  dst = adj[1].astype(jnp.int32)
  partials = _sc_aggregate(input, src, dst)
  return _tc_combine_matmul_relu(partials, weight, input.shape[0])
